# baseline (speedup 1.0000x reference)
import functools

import jax
import jax.numpy as jnp
from jax.experimental import pallas as pl
from jax.experimental.pallas import tpu as pltpu

RADIUS = 2.5
TQ = 256
BIG = 1e30


def _kth_threshold(d, k):
    m = jnp.min(d, axis=1, keepdims=True)
    m0 = m
    for _ in range(k - 1):
        m = jnp.min(jnp.where(d > m, d, BIG), axis=1, keepdims=True)
    return m0, m


def _lp(x):
    return x.astype(jnp.bfloat16).astype(jnp.float32)


def _dist_block(q, ref_t):
    q0 = q[:, 0:1]
    q1 = q[:, 1:2]
    q2 = q[:, 2:3]
    r0 = ref_t[0:1, :]
    r1 = ref_t[1:2, :]
    r2 = ref_t[2:3, :]
    dot = _lp(q0) * _lp(r0) + _lp(q1) * _lp(r1) + _lp(q2) * _lp(r2)
    qq = q0 * q0 + q1 * q1 + q2 * q2
    rr = r0 * r0 + r1 * r1 + r2 * r2
    return qq + rr - 2.0 * dot


def _select_weights(d, k):
    m0, theta = _kth_threshold(d, k)
    w_in = (d <= jnp.minimum(theta, RADIUS)).astype(jnp.float32)
    cnt_out = float(k) - jnp.sum(w_in, axis=1, keepdims=True)
    mask0 = (d == m0).astype(jnp.float32)
    inv_c0 = 1.0 / jnp.sum(mask0, axis=1, keepdims=True)
    return w_in + (cnt_out * inv_c0) * mask0


def _wsum(w, pts):
    return jax.lax.dot_general(
        w, pts, (((1,), (0,)), ((), ())),
        precision=jax.lax.Precision.HIGHEST,
        preferred_element_type=jnp.float32)


def _curv_kernel(ptq_ref, ptt_ref, ptf_ref, psq_ref, pst_ref,
                 warpq_ref, warpf_ref, curv2_ref, curv1_ref, *, n, k):
    q = ptq_ref[0]
    d = _dist_block(q, ptt_ref[0])
    w = _select_weights(d, k)
    curv2_ref[0] = (_wsum(w, ptf_ref[0]) - float(k) * q) / 9.0

    d = _dist_block(psq_ref[0], pst_ref[0])
    w = _select_weights(d, k)
    curv1_ref[0] = (_wsum(w, warpf_ref[0]) - float(k) * warpq_ref[0]) / 9.0


def _interp_kernel(warpq_ref, ptt_ref, curv2_ref, curv1_ref, loss_ref,
                   *, n, k):
    q = warpq_ref[0]
    d = _dist_block(q, ptt_ref[0])

    m0, theta = _kth_threshold(d, k)
    sel = d <= theta
    uv = jnp.where(sel, 1.0 / (d + 1e-8), 0.0)
    norm = jnp.sum(uv, axis=1, keepdims=True)
    a_in = jnp.where(d <= jnp.minimum(theta, RADIUS), uv, 0.0)
    u_out = jnp.sum(uv - a_in, axis=1, keepdims=True)
    mask0 = (d == m0).astype(jnp.float32)
    inv_c0 = 1.0 / jnp.sum(mask0, axis=1, keepdims=True)
    a = a_in + (u_out * inv_c0) * mask0

    inter = _wsum(a, curv2_ref[0]) / norm
    diff = inter - curv1_ref[0]
    loss_ref[0, :, 0] = jnp.sum(diff * diff, axis=1)


@jax.jit
def kernel(pc_source, pc_target, pred_flow):
    b, n, _ = pc_source.shape
    nt = n // TQ
    warp = pc_source + pred_flow
    pt_t = jnp.transpose(pc_target, (0, 2, 1))
    ps_t = jnp.transpose(pc_source, (0, 2, 1))

    q_spec = pl.BlockSpec((1, TQ, 3), lambda bi, ti: (bi, ti, 0))
    fullq_spec = pl.BlockSpec((1, n, 3), lambda bi, ti: (bi, 0, 0))
    fullt_spec = pl.BlockSpec((1, 3, n), lambda bi, ti: (bi, 0, 0))

    curv2, curv1 = pl.pallas_call(
        functools.partial(_curv_kernel, n=n, k=10),
        grid=(b, nt),
        in_specs=[q_spec, fullt_spec, fullq_spec, q_spec, fullt_spec,
                  q_spec, fullq_spec],
        out_specs=[q_spec, q_spec],
        out_shape=[jax.ShapeDtypeStruct((b, n, 3), jnp.float32),
                   jax.ShapeDtypeStruct((b, n, 3), jnp.float32)],
        compiler_params=pltpu.CompilerParams(
            dimension_semantics=("parallel", "parallel")),
    )(pc_target, pt_t, pc_target, pc_source, ps_t, warp, warp)

    loss_terms = pl.pallas_call(
        functools.partial(_interp_kernel, n=n, k=5),
        grid=(b, nt),
        in_specs=[q_spec, fullt_spec, fullq_spec, q_spec],
        out_specs=pl.BlockSpec((1, TQ, 1), lambda bi, ti: (bi, ti, 0)),
        out_shape=jax.ShapeDtypeStruct((b, n, 1), jnp.float32),
        compiler_params=pltpu.CompilerParams(
            dimension_semantics=("parallel", "parallel")),
    )(warp, pt_t, curv2, curv1)

    return jnp.sum(loss_terms) / b


# speedup vs baseline: 1.2784x; 1.2784x over previous
import functools

import jax
import jax.numpy as jnp
from jax.experimental import pallas as pl
from jax.experimental.pallas import tpu as pltpu

RADIUS = 2.5
TQ = 256
BIG = 1e30


def _kth_threshold(d, k):
    m = jnp.min(d, axis=1, keepdims=True)
    m0 = m
    for _ in range(k - 1):
        m = jnp.min(jnp.where(d > m, d, BIG), axis=1, keepdims=True)
    return m0, m


def _lp(x):
    return x.astype(jnp.bfloat16).astype(jnp.float32)


def _dist_block(q, ref_t):
    q0 = q[:, 0:1]
    q1 = q[:, 1:2]
    q2 = q[:, 2:3]
    r0 = ref_t[0:1, :]
    r1 = ref_t[1:2, :]
    r2 = ref_t[2:3, :]
    dot = _lp(q0) * _lp(r0) + _lp(q1) * _lp(r1) + _lp(q2) * _lp(r2)
    qq = q0 * q0 + q1 * q1 + q2 * q2
    rr = r0 * r0 + r1 * r1 + r2 * r2
    return qq + rr - 2.0 * dot


def _select_weights(d, k):
    m0, theta = _kth_threshold(d, k)
    w_in = (d <= jnp.minimum(theta, RADIUS)).astype(jnp.float32)
    cnt_out = float(k) - jnp.sum(w_in, axis=1, keepdims=True)
    mask0 = (d == m0).astype(jnp.float32)
    inv_c0 = 1.0 / jnp.sum(mask0, axis=1, keepdims=True)
    return w_in + (cnt_out * inv_c0) * mask0


def _curv_kernel(ptq_ref, ptt_ref, psq_ref, pst_ref, warpq_ref, warpt_ref,
                 curv2_ref, curv1_ref, *, n, k):
    q = ptq_ref[0]
    ref_t = ptt_ref[0]
    d = _dist_block(q, ref_t)
    w = _select_weights(d, k)
    rows = []
    for c in range(3):
        s = jnp.sum(w * ref_t[c:c+1, :], axis=1)
        rows.append((s - float(k) * q[:, c]) / 9.0)
    curv2_ref[0] = jnp.stack(rows, axis=0)

    q = psq_ref[0]
    ref_t = pst_ref[0]
    wq = warpq_ref[0]
    wt = warpt_ref[0]
    d = _dist_block(q, ref_t)
    w = _select_weights(d, k)
    rows = []
    for c in range(3):
        s = jnp.sum(w * wt[c:c+1, :], axis=1)
        rows.append((s - float(k) * wq[:, c]) / 9.0)
    curv1_ref[0] = jnp.stack(rows, axis=0)


def _interp_kernel(warpq_ref, ptt_ref, curv2t_ref, curv1t_ref, loss_ref,
                   *, n, k):
    q = warpq_ref[0]
    ref_t = ptt_ref[0]
    d = _dist_block(q, ref_t)

    m0, theta = _kth_threshold(d, k)
    sel = d <= theta
    uv = jnp.where(sel, 1.0 / (d + 1e-8), 0.0)
    norm = jnp.sum(uv, axis=1, keepdims=True)
    a_in = jnp.where(d <= jnp.minimum(theta, RADIUS), uv, 0.0)
    u_out = jnp.sum(uv - a_in, axis=1, keepdims=True)
    mask0 = (d == m0).astype(jnp.float32)
    inv_c0 = 1.0 / jnp.sum(mask0, axis=1, keepdims=True)
    a = a_in + (u_out * inv_c0) * mask0

    c2 = curv2t_ref[0]
    c1 = curv1t_ref[0]
    acc = jnp.zeros((TQ,), jnp.float32)
    for c in range(3):
        inter = jnp.sum(a * c2[c:c+1, :], axis=1) / norm[:, 0]
        diff = inter - c1[c, :]
        acc = acc + diff * diff
    loss_ref[0, :, 0] = acc


@jax.jit
def kernel(pc_source, pc_target, pred_flow):
    b, n, _ = pc_source.shape
    nt = n // TQ
    warp = pc_source + pred_flow
    pt_t = jnp.transpose(pc_target, (0, 2, 1))
    ps_t = jnp.transpose(pc_source, (0, 2, 1))
    warp_t = jnp.transpose(warp, (0, 2, 1))

    q_spec = pl.BlockSpec((1, TQ, 3), lambda bi, ti: (bi, ti, 0))
    full_spec = pl.BlockSpec((1, 3, n), lambda bi, ti: (bi, 0, 0))
    out_spec = pl.BlockSpec((1, 3, TQ), lambda bi, ti: (bi, 0, ti))

    curv2_t, curv1_t = pl.pallas_call(
        functools.partial(_curv_kernel, n=n, k=10),
        grid=(b, nt),
        in_specs=[q_spec, full_spec, q_spec, full_spec, q_spec, full_spec],
        out_specs=[out_spec, out_spec],
        out_shape=[jax.ShapeDtypeStruct((b, 3, n), jnp.float32),
                   jax.ShapeDtypeStruct((b, 3, n), jnp.float32)],
        compiler_params=pltpu.CompilerParams(
            dimension_semantics=("parallel", "parallel")),
    )(pc_target, pt_t, pc_source, ps_t, warp, warp_t)

    loss_terms = pl.pallas_call(
        functools.partial(_interp_kernel, n=n, k=5),
        grid=(b, nt),
        in_specs=[q_spec, full_spec, full_spec, out_spec],
        out_specs=pl.BlockSpec((1, TQ, 1), lambda bi, ti: (bi, ti, 0)),
        out_shape=jax.ShapeDtypeStruct((b, n, 1), jnp.float32),
        compiler_params=pltpu.CompilerParams(
            dimension_semantics=("parallel", "parallel")),
    )(warp, pt_t, curv2_t, curv1_t)

    return jnp.sum(loss_terms) / b


# speedup vs baseline: 1.2900x; 1.0090x over previous
import functools

import jax
import jax.numpy as jnp
from jax.experimental import pallas as pl
from jax.experimental.pallas import tpu as pltpu

RADIUS = 2.5
TQ = 256
BIG = 1e30


def _kth_threshold(d, k):
    h = d.shape[1] // 2
    a = jnp.minimum(d[:, :h], d[:, h:])
    b = jnp.maximum(d[:, :h], d[:, h:])
    m = jnp.min(a, axis=1, keepdims=True)
    m0 = m
    for t in range(k - 1):
        sel = a == m
        a = jnp.where(sel, b, a)
        if t != k - 2:
            b = jnp.where(sel, BIG, b)
        m = jnp.min(a, axis=1, keepdims=True)
    return m0, m


def _lp(x):
    return x.astype(jnp.bfloat16).astype(jnp.float32)


def _dist_block(q, ref_t):
    q0 = q[:, 0:1]
    q1 = q[:, 1:2]
    q2 = q[:, 2:3]
    r0 = ref_t[0:1, :]
    r1 = ref_t[1:2, :]
    r2 = ref_t[2:3, :]
    dot = _lp(q0) * _lp(r0) + _lp(q1) * _lp(r1) + _lp(q2) * _lp(r2)
    qq = q0 * q0 + q1 * q1 + q2 * q2
    rr = r0 * r0 + r1 * r1 + r2 * r2
    return qq + rr - 2.0 * dot


def _select_weights(d, k):
    m0, theta = _kth_threshold(d, k)
    w_in = (d <= jnp.minimum(theta, RADIUS)).astype(jnp.float32)
    cnt_out = float(k) - jnp.sum(w_in, axis=1, keepdims=True)
    mask0 = (d == m0).astype(jnp.float32)
    inv_c0 = 1.0 / jnp.sum(mask0, axis=1, keepdims=True)
    return w_in + (cnt_out * inv_c0) * mask0


def _curv_kernel(ptq_ref, ptt_ref, psq_ref, pst_ref, warpq_ref, warpt_ref,
                 curv2_ref, curv1_ref, *, n, k):
    q = ptq_ref[0]
    ref_t = ptt_ref[0]
    d = _dist_block(q, ref_t)
    w = _select_weights(d, k)
    rows = []
    for c in range(3):
        s = jnp.sum(w * ref_t[c:c+1, :], axis=1)
        rows.append((s - float(k) * q[:, c]) / 9.0)
    curv2_ref[0] = jnp.stack(rows, axis=0)

    q = psq_ref[0]
    ref_t = pst_ref[0]
    wq = warpq_ref[0]
    wt = warpt_ref[0]
    d = _dist_block(q, ref_t)
    w = _select_weights(d, k)
    rows = []
    for c in range(3):
        s = jnp.sum(w * wt[c:c+1, :], axis=1)
        rows.append((s - float(k) * wq[:, c]) / 9.0)
    curv1_ref[0] = jnp.stack(rows, axis=0)


def _interp_kernel(warpq_ref, ptt_ref, curv2t_ref, curv1t_ref, loss_ref,
                   *, n, k):
    q = warpq_ref[0]
    ref_t = ptt_ref[0]
    d = _dist_block(q, ref_t)

    m0, theta = _kth_threshold(d, k)
    sel = d <= theta
    uv = jnp.where(sel, 1.0 / (d + 1e-8), 0.0)
    norm = jnp.sum(uv, axis=1, keepdims=True)
    a_in = jnp.where(d <= jnp.minimum(theta, RADIUS), uv, 0.0)
    u_out = jnp.sum(uv - a_in, axis=1, keepdims=True)
    mask0 = (d == m0).astype(jnp.float32)
    inv_c0 = 1.0 / jnp.sum(mask0, axis=1, keepdims=True)
    a = a_in + (u_out * inv_c0) * mask0

    c2 = curv2t_ref[0]
    c1 = curv1t_ref[0]
    acc = jnp.zeros((TQ,), jnp.float32)
    for c in range(3):
        inter = jnp.sum(a * c2[c:c+1, :], axis=1) / norm[:, 0]
        diff = inter - c1[c, :]
        acc = acc + diff * diff
    loss_ref[0, :, 0] = acc


@jax.jit
def kernel(pc_source, pc_target, pred_flow):
    b, n, _ = pc_source.shape
    nt = n // TQ
    warp = pc_source + pred_flow
    pt_t = jnp.transpose(pc_target, (0, 2, 1))
    ps_t = jnp.transpose(pc_source, (0, 2, 1))
    warp_t = jnp.transpose(warp, (0, 2, 1))

    q_spec = pl.BlockSpec((1, TQ, 3), lambda bi, ti: (bi, ti, 0))
    full_spec = pl.BlockSpec((1, 3, n), lambda bi, ti: (bi, 0, 0))
    out_spec = pl.BlockSpec((1, 3, TQ), lambda bi, ti: (bi, 0, ti))

    curv2_t, curv1_t = pl.pallas_call(
        functools.partial(_curv_kernel, n=n, k=10),
        grid=(b, nt),
        in_specs=[q_spec, full_spec, q_spec, full_spec, q_spec, full_spec],
        out_specs=[out_spec, out_spec],
        out_shape=[jax.ShapeDtypeStruct((b, 3, n), jnp.float32),
                   jax.ShapeDtypeStruct((b, 3, n), jnp.float32)],
        compiler_params=pltpu.CompilerParams(
            dimension_semantics=("parallel", "parallel")),
    )(pc_target, pt_t, pc_source, ps_t, warp, warp_t)

    loss_terms = pl.pallas_call(
        functools.partial(_interp_kernel, n=n, k=5),
        grid=(b, nt),
        in_specs=[q_spec, full_spec, full_spec, out_spec],
        out_specs=pl.BlockSpec((1, TQ, 1), lambda bi, ti: (bi, ti, 0)),
        out_shape=jax.ShapeDtypeStruct((b, n, 1), jnp.float32),
        compiler_params=pltpu.CompilerParams(
            dimension_semantics=("parallel", "parallel")),
    )(warp, pt_t, curv2_t, curv1_t)

    return jnp.sum(loss_terms) / b


# speedup vs baseline: 1.3080x; 1.0140x over previous
import functools

import jax
import jax.numpy as jnp
from jax.experimental import pallas as pl
from jax.experimental.pallas import tpu as pltpu

RADIUS = 2.5
TQ = 512
BIG = 1e30


def _kth_threshold(d, k):
    h = d.shape[1] // 2
    a = jnp.minimum(d[:, :h], d[:, h:])
    b = jnp.maximum(d[:, :h], d[:, h:])
    m = jnp.min(a, axis=1, keepdims=True)
    m0 = m
    for t in range(k - 1):
        sel = a == m
        a = jnp.where(sel, b, a)
        if t != k - 2:
            b = jnp.where(sel, BIG, b)
        m = jnp.min(a, axis=1, keepdims=True)
    return m0, m


def _lp(x):
    return x.astype(jnp.bfloat16).astype(jnp.float32)


def _dist_block(q, ref_t):
    q0 = q[:, 0:1]
    q1 = q[:, 1:2]
    q2 = q[:, 2:3]
    r0 = ref_t[0:1, :]
    r1 = ref_t[1:2, :]
    r2 = ref_t[2:3, :]
    dot = _lp(q0) * _lp(r0) + _lp(q1) * _lp(r1) + _lp(q2) * _lp(r2)
    qq = q0 * q0 + q1 * q1 + q2 * q2
    rr = r0 * r0 + r1 * r1 + r2 * r2
    return qq + rr - 2.0 * dot


def _select_weights(d, k):
    m0, theta = _kth_threshold(d, k)
    w_in = (d <= jnp.minimum(theta, RADIUS)).astype(jnp.float32)
    cnt_out = float(k) - jnp.sum(w_in, axis=1, keepdims=True)
    mask0 = (d == m0).astype(jnp.float32)
    inv_c0 = 1.0 / jnp.sum(mask0, axis=1, keepdims=True)
    return w_in + (cnt_out * inv_c0) * mask0


def _curv_kernel(ptq_ref, ptt_ref, psq_ref, pst_ref, warpq_ref, warpt_ref,
                 curv2_ref, curv1_ref, *, n, k):
    q = ptq_ref[0]
    ref_t = ptt_ref[0]
    d = _dist_block(q, ref_t)
    w = _select_weights(d, k)
    rows = []
    for c in range(3):
        s = jnp.sum(w * ref_t[c:c+1, :], axis=1)
        rows.append((s - float(k) * q[:, c]) / 9.0)
    curv2_ref[0] = jnp.stack(rows, axis=0)

    q = psq_ref[0]
    ref_t = pst_ref[0]
    wq = warpq_ref[0]
    wt = warpt_ref[0]
    d = _dist_block(q, ref_t)
    w = _select_weights(d, k)
    rows = []
    for c in range(3):
        s = jnp.sum(w * wt[c:c+1, :], axis=1)
        rows.append((s - float(k) * wq[:, c]) / 9.0)
    curv1_ref[0] = jnp.stack(rows, axis=0)


def _interp_kernel(warpq_ref, ptt_ref, curv2t_ref, curv1t_ref, loss_ref,
                   *, n, k):
    q = warpq_ref[0]
    ref_t = ptt_ref[0]
    d = _dist_block(q, ref_t)

    m0, theta = _kth_threshold(d, k)
    sel = d <= theta
    uv = jnp.where(sel, 1.0 / (d + 1e-8), 0.0)
    norm = jnp.sum(uv, axis=1, keepdims=True)
    a_in = jnp.where(d <= jnp.minimum(theta, RADIUS), uv, 0.0)
    u_out = jnp.sum(uv - a_in, axis=1, keepdims=True)
    mask0 = (d == m0).astype(jnp.float32)
    inv_c0 = 1.0 / jnp.sum(mask0, axis=1, keepdims=True)
    a = a_in + (u_out * inv_c0) * mask0

    c2 = curv2t_ref[0]
    c1 = curv1t_ref[0]
    acc = jnp.zeros((TQ,), jnp.float32)
    for c in range(3):
        inter = jnp.sum(a * c2[c:c+1, :], axis=1) / norm[:, 0]
        diff = inter - c1[c, :]
        acc = acc + diff * diff
    loss_ref[0, :, 0] = acc


@jax.jit
def kernel(pc_source, pc_target, pred_flow):
    b, n, _ = pc_source.shape
    nt = n // TQ
    warp = pc_source + pred_flow
    pt_t = jnp.transpose(pc_target, (0, 2, 1))
    ps_t = jnp.transpose(pc_source, (0, 2, 1))
    warp_t = jnp.transpose(warp, (0, 2, 1))

    q_spec = pl.BlockSpec((1, TQ, 3), lambda bi, ti: (bi, ti, 0))
    full_spec = pl.BlockSpec((1, 3, n), lambda bi, ti: (bi, 0, 0))
    out_spec = pl.BlockSpec((1, 3, TQ), lambda bi, ti: (bi, 0, ti))

    curv2_t, curv1_t = pl.pallas_call(
        functools.partial(_curv_kernel, n=n, k=10),
        grid=(b, nt),
        in_specs=[q_spec, full_spec, q_spec, full_spec, q_spec, full_spec],
        out_specs=[out_spec, out_spec],
        out_shape=[jax.ShapeDtypeStruct((b, 3, n), jnp.float32),
                   jax.ShapeDtypeStruct((b, 3, n), jnp.float32)],
        compiler_params=pltpu.CompilerParams(
            dimension_semantics=("parallel", "parallel")),
    )(pc_target, pt_t, pc_source, ps_t, warp, warp_t)

    loss_terms = pl.pallas_call(
        functools.partial(_interp_kernel, n=n, k=5),
        grid=(b, nt),
        in_specs=[q_spec, full_spec, full_spec, out_spec],
        out_specs=pl.BlockSpec((1, TQ, 1), lambda bi, ti: (bi, ti, 0)),
        out_shape=jax.ShapeDtypeStruct((b, n, 1), jnp.float32),
        compiler_params=pltpu.CompilerParams(
            dimension_semantics=("parallel", "parallel")),
    )(warp, pt_t, curv2_t, curv1_t)

    return jnp.sum(loss_terms) / b
